# Initial kernel scaffold; baseline (speedup 1.0000x reference)
import jax
import jax.numpy as jnp
from jax.experimental import pallas as pl


def kernel(cat, col, fab, cat_table, col_table, fab_table, W, b):
    raise NotImplementedError("write your pallas kernel here")


# speedup vs baseline: 1.5862x; 1.5862x over previous
import functools

import jax
import jax.numpy as jnp
from jax import lax
from jax.experimental import pallas as pl
from jax.experimental.pallas import tpu as pltpu
from jax.experimental.pallas import tpu_sc as plsc

B = 16384
D = 64

_info = plsc.get_sparse_core_info()
_NC, _NS = _info.num_cores, _info.num_subcores
_NW = _NC * _NS
_BPW = B // _NW
_CHUNK = 128
_NCHUNK = _BPW // _CHUNK


def _gather3_body(cat_i, col_i, fab_i, cat_t, col_t, fab_t,
                  o_cat, o_col, o_fab,
                  iv0, iv1, iv2, rv0, rv1, rv2, sem):
    wid = lax.axis_index("s") * _NC + lax.axis_index("c")
    base = wid * _BPW
    pltpu.sync_copy(cat_i.at[pl.ds(base, _BPW)], iv0)
    pltpu.sync_copy(col_i.at[pl.ds(base, _BPW)], iv1)
    pltpu.sync_copy(fab_i.at[pl.ds(base, _BPW)], iv2)
    copies = []
    for iv, tab, rv in ((iv0, cat_t, rv0), (iv1, col_t, rv1), (iv2, fab_t, rv2)):
        for j in range(_NCHUNK):
            sl = pl.ds(j * _CHUNK, _CHUNK)
            copies.append(
                pltpu.async_copy(tab.at[iv.at[sl]], rv.at[sl], sem))
    for c in copies:
        c.wait()
    pltpu.sync_copy(rv0, o_cat.at[pl.ds(base, _BPW)])
    pltpu.sync_copy(rv1, o_col.at[pl.ds(base, _BPW)])
    pltpu.sync_copy(rv2, o_fab.at[pl.ds(base, _BPW)])


@jax.jit
def _gather3(cat, col, fab, cat_table, col_table, fab_table):
    mesh = plsc.VectorSubcoreMesh(core_axis_name="c", subcore_axis_name="s")
    f = functools.partial(
        pl.kernel,
        mesh=mesh,
        out_type=[jax.ShapeDtypeStruct((B, D), jnp.float32)] * 3,
        scratch_types=[pltpu.VMEM((_BPW,), jnp.int32)] * 3
        + [pltpu.VMEM((_BPW, D), jnp.float32)] * 3
        + [pltpu.SemaphoreType.DMA],
        compiler_params=pltpu.CompilerParams(use_tc_tiling_on_sc=False),
    )(_gather3_body)
    return f(cat, col, fab, cat_table, col_table, fab_table)


def _fuse_body(x0_ref, x1_ref, x2_ref, wt_ref, b_ref, o_ref):
    wt = wt_ref[...]
    acc = jnp.dot(x0_ref[...], wt[0:D, :], preferred_element_type=jnp.float32)
    acc += jnp.dot(x1_ref[...], wt[D:2 * D, :], preferred_element_type=jnp.float32)
    acc += jnp.dot(x2_ref[...], wt[2 * D:3 * D, :], preferred_element_type=jnp.float32)
    o_ref[...] = acc + b_ref[...]


_BLK = 2048


@jax.jit
def _fuse(x0, x1, x2, wt, b2):
    grid = (B // _BLK,)
    return pl.pallas_call(
        _fuse_body,
        grid=grid,
        in_specs=[
            pl.BlockSpec((_BLK, D), lambda i: (i, 0)),
            pl.BlockSpec((_BLK, D), lambda i: (i, 0)),
            pl.BlockSpec((_BLK, D), lambda i: (i, 0)),
            pl.BlockSpec((3 * D, D), lambda i: (0, 0)),
            pl.BlockSpec((1, D), lambda i: (0, 0)),
        ],
        out_specs=pl.BlockSpec((_BLK, D), lambda i: (i, 0)),
        out_shape=jax.ShapeDtypeStruct((B, D), jnp.float32),
    )(x0, x1, x2, wt, b2)


def kernel(cat, col, fab, cat_table, col_table, fab_table, W, b):
    cat_emb, col_emb, fab_emb = _gather3(
        cat.astype(jnp.int32), col.astype(jnp.int32), fab.astype(jnp.int32),
        cat_table, col_table, fab_table)
    return _fuse(cat_emb, col_emb, fab_emb, W.T, b.reshape(1, D))


# speedup vs baseline: 1.8055x; 1.1383x over previous
import functools

import jax
import jax.numpy as jnp
from jax import lax
from jax.experimental import pallas as pl
from jax.experimental.pallas import tpu as pltpu
from jax.experimental.pallas import tpu_sc as plsc

B = 16384
D = 64

_info = plsc.get_sparse_core_info()
_NC, _NS = _info.num_cores, _info.num_subcores
_NW = _NC * _NS
_BPW = B // _NW
_CHUNK = 128
_NCHUNK = _BPW // _CHUNK


def _gather3_body(cat_i, col_i, fab_i, cat_t, col_t, fab_t,
                  o_cat, o_col, o_fab,
                  iv0, iv1, iv2, rv0, rv1, rv2, sem):
    wid = lax.axis_index("s") * _NC + lax.axis_index("c")
    base = wid * _BPW
    pltpu.sync_copy(cat_i.at[pl.ds(base, _BPW)], iv0)
    pltpu.sync_copy(col_i.at[pl.ds(base, _BPW)], iv1)
    pltpu.sync_copy(fab_i.at[pl.ds(base, _BPW)], iv2)
    copies = []
    for iv, tab, rv in ((iv0, cat_t, rv0), (iv1, col_t, rv1), (iv2, fab_t, rv2)):
        for j in range(_NCHUNK):
            sl = pl.ds(j * _CHUNK, _CHUNK)
            copies.append(
                pltpu.async_copy(tab.at[iv.at[sl]], rv.at[sl], sem))
    for c in copies:
        c.wait()
    pltpu.sync_copy(rv0, o_cat.at[pl.ds(base, _BPW), pl.ds(0, D)])
    pltpu.sync_copy(rv1, o_col.at[pl.ds(base, _BPW), pl.ds(0, D)])
    pltpu.sync_copy(rv2, o_fab.at[pl.ds(base, _BPW), pl.ds(0, D)])


@jax.jit
def _gather3(cat, col, fab, cat_table, col_table, fab_table):
    mesh = plsc.VectorSubcoreMesh(core_axis_name="c", subcore_axis_name="s")
    f = functools.partial(
        pl.kernel,
        mesh=mesh,
        out_type=[jax.ShapeDtypeStruct((B, 2 * D), jnp.float32)] * 3,
        scratch_types=[pltpu.VMEM((_BPW,), jnp.int32)] * 3
        + [pltpu.VMEM((_BPW, D), jnp.float32)] * 3
        + [pltpu.SemaphoreType.DMA],
        compiler_params=pltpu.CompilerParams(use_tc_tiling_on_sc=False),
    )(_gather3_body)
    return f(cat, col, fab, cat_table, col_table, fab_table)


def _fuse_body(x0_ref, x1_ref, x2_ref, wt_ref, b_ref, o_ref):
    wt = wt_ref[...]
    acc = jnp.dot(x0_ref[:, :D], wt[0:D, :], preferred_element_type=jnp.float32)
    acc += jnp.dot(x1_ref[:, :D], wt[D:2 * D, :], preferred_element_type=jnp.float32)
    acc += jnp.dot(x2_ref[:, :D], wt[2 * D:3 * D, :], preferred_element_type=jnp.float32)
    o_ref[...] = acc + b_ref[...]


_BLK = 2048


@jax.jit
def _fuse(x0, x1, x2, wt, b2):
    grid = (B // _BLK,)
    return pl.pallas_call(
        _fuse_body,
        grid=grid,
        in_specs=[
            pl.BlockSpec((_BLK, 2 * D), lambda i: (i, 0)),
            pl.BlockSpec((_BLK, 2 * D), lambda i: (i, 0)),
            pl.BlockSpec((_BLK, 2 * D), lambda i: (i, 0)),
            pl.BlockSpec((3 * D, D), lambda i: (0, 0)),
            pl.BlockSpec((1, D), lambda i: (0, 0)),
        ],
        out_specs=pl.BlockSpec((_BLK, D), lambda i: (i, 0)),
        out_shape=jax.ShapeDtypeStruct((B, D), jnp.float32),
    )(x0, x1, x2, wt, b2)


def kernel(cat, col, fab, cat_table, col_table, fab_table, W, b):
    cat_emb, col_emb, fab_emb = _gather3(
        cat.astype(jnp.int32), col.astype(jnp.int32), fab.astype(jnp.int32),
        cat_table, col_table, fab_table)
    return _fuse(cat_emb, col_emb, fab_emb, W.T, b.reshape(1, D))
